# baseline (speedup 1.0000x reference)
import functools

import jax
import jax.numpy as jnp
from jax import lax
from jax.experimental import pallas as pl
from jax.experimental.pallas import tpu as pltpu
from jax.experimental.pallas import tpu_sc as plsc

N = 10000
E = 320000
D = 128
DW = 64
NC = 2
NS = 16
CHUNK = 128
CH = 80
EPT = CH * CHUNK
E_PAD = NC * NS * EPT
RPT = 640
R = NS * RPT
JUNK = N
NBUF = 4
HALF = CH // 2
EPS = 1e-5

_MESH = plsc.VectorSubcoreMesh(core_axis_name="c", subcore_axis_name="s")


@functools.partial(
    pl.kernel,
    out_type=(jax.ShapeDtypeStruct((NC * R, DW), jnp.float32),
              jax.ShapeDtypeStruct((NC * R, DW), jnp.float32)),
    mesh=_MESH,
    scratch_types=[
        pltpu.VMEM((HALF, CHUNK), jnp.int32),
        pltpu.VMEM((HALF, CHUNK), jnp.int32),
        *[pltpu.VMEM((CHUNK, DW), jnp.float32) for _ in range(NBUF)],
        pltpu.VMEM_SHARED((N, DW), jnp.float32),
        pltpu.VMEM_SHARED((R, DW), jnp.float32),
        *[pltpu.SemaphoreType.DMA for _ in range(NBUF)],
    ],
    compiler_params=pltpu.CompilerParams(use_tc_tiling_on_sc=False),
)
def _sc_scatter(srca_hbm, srcb_hbm, r_hbm, c_hbm, pa_hbm, pb_hbm,
                r_v, c_v, *rest):
    bufs = rest[:NBUF]
    xs_sp = rest[NBUF]
    acc = rest[NBUF + 1]
    sems = rest[NBUF + 2:]
    cid = lax.axis_index("c")
    sid = lax.axis_index("s")
    wid = cid * NS + sid

    for src_hbm, out_hbm in ((srca_hbm, pa_hbm), (srcb_hbm, pb_hbm)):
        @pl.when(sid == 0)
        def _stage():
            pltpu.sync_copy(src_hbm, xs_sp)

        z = bufs[0]

        def zrow(i, carry):
            for jj in range(DW // 16):
                z[i, pl.ds(jj * 16, 16)] = jnp.zeros((16,), jnp.float32)
            return carry

        lax.fori_loop(0, CHUNK, zrow, 0)
        for kk in range(RPT // CHUNK):
            pltpu.sync_copy(z, acc.at[pl.ds(sid * RPT + kk * CHUNK, CHUNK)])
        plsc.subcore_barrier()

        for h in range(2):
            base = wid * CH + h * HALF
            pltpu.sync_copy(r_hbm.at[pl.ds(base, HALF)], r_v)
            pltpu.sync_copy(c_hbm.at[pl.ds(base, HALF)], c_v)

            for b in range(NBUF):
                pltpu.async_copy(xs_sp.at[r_v.at[b]], bufs[b], sems[b])

            def step(t, carry):
                j0 = t * NBUF
                for b in range(NBUF):
                    j = j0 + b
                    pltpu.make_async_copy(
                        xs_sp.at[r_v.at[j]], bufs[b], sems[b]).wait()
                    pltpu.sync_copy(bufs[b], acc.at[c_v.at[j]], add=True)
                    pltpu.async_copy(
                        xs_sp.at[r_v.at[j + NBUF]], bufs[b], sems[b])
                return carry

            lax.fori_loop(0, (HALF - NBUF) // NBUF, step, 0)
            for b in range(NBUF):
                j = HALF - NBUF + b
                pltpu.make_async_copy(
                    xs_sp.at[r_v.at[j]], bufs[b], sems[b]).wait()
                pltpu.sync_copy(bufs[b], acc.at[c_v.at[j]], add=True)

        plsc.subcore_barrier()
        pltpu.sync_copy(acc.at[pl.ds(sid * RPT, RPT)],
                        out_hbm.at[pl.ds(cid * R + sid * RPT, RPT)])
        plsc.subcore_barrier()


@functools.partial(
    pl.kernel,
    out_type=jax.ShapeDtypeStruct((NC * NS, R), jnp.float32),
    mesh=_MESH,
    scratch_types=[
        pltpu.VMEM((CH, CHUNK), jnp.int32),
        pltpu.VMEM((R,), jnp.float32),
    ],
    compiler_params=pltpu.CompilerParams(needs_layout_passes=False),
)
def _sc_degree(c_hbm, out_hbm, c_v, hist):
    cid = lax.axis_index("c")
    sid = lax.axis_index("s")
    wid = cid * NS + sid
    pltpu.sync_copy(c_hbm.at[pl.ds(wid * CH, CH)], c_v)

    def zero(i, carry):
        hist[pl.ds(i * 16, 16)] = jnp.zeros((16,), jnp.float32)
        return carry

    lax.fori_loop(0, R // 16, zero, 0)
    ones = jnp.ones((16,), jnp.float32)

    def step(j, carry):
        for k in range(CHUNK // 16):
            idx = c_v[j, pl.ds(k * 16, 16)]
            plsc.addupdate_scatter(hist, [idx], ones)
        return carry

    lax.fori_loop(0, CH, step, 0)
    pltpu.sync_copy(hist, out_hbm.at[wid])


def _k0(histT, x, W1):
    def body(h_ref, x_ref, w_ref, dinv_ref, xsa_ref, xsb_ref):
        deg = jnp.sum(h_ref[...], axis=1, keepdims=True)
        dinv = lax.rsqrt(deg[0:N] + 1.0)
        dinv_ref[...] = dinv
        xw = jnp.dot(x_ref[...], w_ref[...], preferred_element_type=jnp.float32)
        xs = dinv * xw
        xsa_ref[...] = xs[:, 0:DW]
        xsb_ref[...] = xs[:, DW:D]

    return pl.pallas_call(
        body,
        out_shape=(jax.ShapeDtypeStruct((N, 1), jnp.float32),
                   jax.ShapeDtypeStruct((N, DW), jnp.float32),
                   jax.ShapeDtypeStruct((N, DW), jnp.float32)),
    )(histT, x, W1)


def _agg_bn(pa_ref, pb_ref, xsa_ref, xsb_ref, dinv_ref, b_ref, g_ref, bt_ref):
    dinv = dinv_ref[...]
    psum = jnp.concatenate(
        [pa_ref[0:N, :] + pa_ref[R:R + N, :] + xsa_ref[...],
         pb_ref[0:N, :] + pb_ref[R:R + N, :] + xsb_ref[...]], axis=1)
    agg = dinv * psum + b_ref[...][None, :]
    mean = jnp.mean(agg, axis=0, keepdims=True)
    var = jnp.mean((agg - mean) ** 2, axis=0, keepdims=True)
    y = g_ref[...][None, :] * (agg - mean) * lax.rsqrt(var + EPS)
    return y + bt_ref[...][None, :], dinv


def _k_layer(pa, pb, xsa, xsb, dinv, b, g, bt, Wn, relu):
    def body(pa_ref, pb_ref, xsa_ref, xsb_ref, dinv_ref, b_ref, g_ref,
             bt_ref, w_ref, xsa_o, xsb_o):
        y, dinv = _agg_bn(pa_ref, pb_ref, xsa_ref, xsb_ref, dinv_ref,
                          b_ref, g_ref, bt_ref)
        if relu:
            y = jnp.maximum(y, 0.0)
        xw = jnp.dot(y, w_ref[...], preferred_element_type=jnp.float32)
        xs = dinv * xw
        xsa_o[...] = xs[:, 0:DW]
        xsb_o[...] = xs[:, DW:D]

    return pl.pallas_call(
        body,
        out_shape=(jax.ShapeDtypeStruct((N, DW), jnp.float32),
                   jax.ShapeDtypeStruct((N, DW), jnp.float32)),
    )(pa, pb, xsa, xsb, dinv, b, g, bt, Wn)


def _k_last(pa, pb, xsa, xsb, dinv, b, g, bt):
    def body(pa_ref, pb_ref, xsa_ref, xsb_ref, dinv_ref, b_ref, g_ref,
             bt_ref, out_ref):
        y, _ = _agg_bn(pa_ref, pb_ref, xsa_ref, xsb_ref, dinv_ref,
                       b_ref, g_ref, bt_ref)
        out_ref[...] = y

    return pl.pallas_call(
        body,
        out_shape=jax.ShapeDtypeStruct((N, D), jnp.float32),
    )(pa, pb, xsa, xsb, dinv, b, g, bt)


def _pad_cols(a, width):
    return jnp.concatenate(
        [a, jnp.zeros(a.shape[:-1] + (width - a.shape[-1],), a.dtype)], axis=-1)


def kernel(x, edge_index,
           We1, be1, g1, bt1, We2, be2, g2, bt2, We3, be3, g3, bt3,
           Wd1, bd1, gd1, btd1, Wd2, bd2, gd2, btd2, Wd3, bd3, gd3, btd3):
    row = edge_index[0].astype(jnp.int32)
    col = edge_index[1].astype(jnp.int32)
    pad = E_PAD - E
    r_idx = jnp.concatenate([row, jnp.zeros((pad,), jnp.int32)])
    c_idx = jnp.concatenate([col, jnp.full((pad,), JUNK, jnp.int32)])
    r_idx = r_idx.reshape(NC * NS * CH, CHUNK)
    c_idx = c_idx.reshape(NC * NS * CH, CHUNK)

    We3p = _pad_cols(We3, D)
    be3p = _pad_cols(be3, D)
    g3p = jnp.concatenate([g3, jnp.ones((D - g3.shape[0],), g3.dtype)])
    bt3p = _pad_cols(bt3, D)
    Wd1p = jnp.concatenate(
        [Wd1, jnp.zeros((D - Wd1.shape[0], Wd1.shape[1]), Wd1.dtype)], axis=0)

    hist = _sc_degree(c_idx)
    histT = jnp.transpose(hist)

    dinv, xsa, xsb = _k0(histT, x, We1)

    layers = [
        (be1, g1, bt1, We2, True),
        (be2, g2, bt2, We3p, True),
        (be3p, g3p, bt3p, Wd1p, False),
        (bd1, gd1, btd1, Wd2, True),
        (bd2, gd2, btd2, Wd3, True),
        (bd3, gd3, btd3, None, False),
    ]
    for b, g, bt, Wn, relu in layers:
        pa, pb = _sc_scatter(xsa, xsb, r_idx, c_idx)
        if Wn is None:
            return _k_last(pa, pb, xsa, xsb, dinv, b, g, bt)
        xsa, xsb = _k_layer(pa, pb, xsa, xsb, dinv, b, g, bt, Wn, relu)


# speedup vs baseline: 1.1331x; 1.1331x over previous
import functools

import jax
import jax.numpy as jnp
from jax import lax
from jax.experimental import pallas as pl
from jax.experimental.pallas import tpu as pltpu
from jax.experimental.pallas import tpu_sc as plsc

N = 10000
E = 320000
D = 128
DW = 64
NC = 2
NS = 16
CHUNK = 128
CH = 80
EPT = CH * CHUNK
E_PAD = NC * NS * EPT
RPT = 640
R = NS * RPT
JUNK = N
NBUF = 4
HALF = CH // 2
EPS = 1e-5

_MESH = plsc.VectorSubcoreMesh(core_axis_name="c", subcore_axis_name="s")


@functools.partial(
    pl.kernel,
    out_type=(jax.ShapeDtypeStruct((NC * R, DW), jnp.float32),
              jax.ShapeDtypeStruct((NC * R, DW), jnp.float32)),
    mesh=_MESH,
    scratch_types=[
        pltpu.VMEM((HALF, CHUNK), jnp.int32),
        pltpu.VMEM((HALF, CHUNK), jnp.int32),
        *[pltpu.VMEM((CHUNK, DW), jnp.float32) for _ in range(NBUF)],
        pltpu.VMEM_SHARED((N, DW), jnp.float32),
        pltpu.VMEM_SHARED((R, DW), jnp.float32),
        *[pltpu.SemaphoreType.DMA for _ in range(2 * NBUF)],
    ],
    compiler_params=pltpu.CompilerParams(use_tc_tiling_on_sc=False),
)
def _sc_scatter(srca_hbm, srcb_hbm, r_hbm, c_hbm, pa_hbm, pb_hbm,
                r_v, c_v, *rest):
    bufs = rest[:NBUF]
    xs_sp = rest[NBUF]
    acc = rest[NBUF + 1]
    gsem = rest[NBUF + 2:2 * NBUF + 2]
    ssem = rest[2 * NBUF + 2:]
    cid = lax.axis_index("c")
    sid = lax.axis_index("s")
    wid = cid * NS + sid

    for src_hbm, out_hbm in ((srca_hbm, pa_hbm), (srcb_hbm, pb_hbm)):
        @pl.when(sid == 0)
        def _stage():
            pltpu.sync_copy(src_hbm, xs_sp)

        z = bufs[0]

        def zrow(i, carry):
            for jj in range(DW // 16):
                z[i, pl.ds(jj * 16, 16)] = jnp.zeros((16,), jnp.float32)
            return carry

        lax.fori_loop(0, CHUNK, zrow, 0)
        for kk in range(RPT // CHUNK):
            pltpu.sync_copy(z, acc.at[pl.ds(sid * RPT + kk * CHUNK, CHUNK)])
        plsc.subcore_barrier()

        def g_start(j, b):
            pltpu.async_copy(xs_sp.at[r_v.at[j]], bufs[b], gsem[b])

        def g_wait(j, b):
            pltpu.make_async_copy(xs_sp.at[r_v.at[j]], bufs[b], gsem[b]).wait()

        def s_start(j, b):
            pltpu.async_copy(bufs[b], acc.at[c_v.at[j]], ssem[b], add=True)

        def s_wait(j, b):
            pltpu.make_async_copy(bufs[b], acc.at[c_v.at[j]], ssem[b]).wait()

        for h in range(2):
            base = wid * CH + h * HALF
            pltpu.sync_copy(r_hbm.at[pl.ds(base, HALF)], r_v)
            pltpu.sync_copy(c_hbm.at[pl.ds(base, HALF)], c_v)

            for m in range(NBUF):
                g_start(m, m)
            for j in range(2):
                g_wait(j, j)
                s_start(j, j)

            def step(t, carry):
                j0 = t * NBUF + 2
                for bp in range(NBUF):
                    j = j0 + bp
                    s_wait(j - 2, bp)
                    g_start(j + 2, bp)
                    b = (bp + 2) % NBUF
                    g_wait(j, b)
                    s_start(j, b)
                return carry

            lax.fori_loop(0, (HALF - 4) // NBUF, step, 0)
            for j in (HALF - 2, HALF - 1):
                s_wait(j - 2, (j - 2) % NBUF)
                g_wait(j, j % NBUF)
                s_start(j, j % NBUF)
            for j in (HALF - 2, HALF - 1):
                s_wait(j, j % NBUF)

        plsc.subcore_barrier()
        pltpu.sync_copy(acc.at[pl.ds(sid * RPT, RPT)],
                        out_hbm.at[pl.ds(cid * R + sid * RPT, RPT)])
        plsc.subcore_barrier()


@functools.partial(
    pl.kernel,
    out_type=jax.ShapeDtypeStruct((NC * NS, R), jnp.float32),
    mesh=_MESH,
    scratch_types=[
        pltpu.VMEM((CH, CHUNK), jnp.int32),
        pltpu.VMEM((R,), jnp.float32),
    ],
    compiler_params=pltpu.CompilerParams(needs_layout_passes=False),
)
def _sc_degree(c_hbm, out_hbm, c_v, hist):
    cid = lax.axis_index("c")
    sid = lax.axis_index("s")
    wid = cid * NS + sid
    pltpu.sync_copy(c_hbm.at[pl.ds(wid * CH, CH)], c_v)

    def zero(i, carry):
        hist[pl.ds(i * 16, 16)] = jnp.zeros((16,), jnp.float32)
        return carry

    lax.fori_loop(0, R // 16, zero, 0)
    ones = jnp.ones((16,), jnp.float32)

    def step(j, carry):
        for k in range(CHUNK // 16):
            idx = c_v[j, pl.ds(k * 16, 16)]
            plsc.addupdate_scatter(hist, [idx], ones)
        return carry

    lax.fori_loop(0, CH, step, 0)
    pltpu.sync_copy(hist, out_hbm.at[wid])


def _k0(histT, x, W1):
    def body(h_ref, x_ref, w_ref, dinv_ref, xsa_ref, xsb_ref):
        deg = jnp.sum(h_ref[...], axis=1, keepdims=True)
        dinv = lax.rsqrt(deg[0:N] + 1.0)
        dinv_ref[...] = dinv
        xw = jnp.dot(x_ref[...], w_ref[...], preferred_element_type=jnp.float32)
        xs = dinv * xw
        xsa_ref[...] = xs[:, 0:DW]
        xsb_ref[...] = xs[:, DW:D]

    return pl.pallas_call(
        body,
        out_shape=(jax.ShapeDtypeStruct((N, 1), jnp.float32),
                   jax.ShapeDtypeStruct((N, DW), jnp.float32),
                   jax.ShapeDtypeStruct((N, DW), jnp.float32)),
    )(histT, x, W1)


def _agg_bn(pa_ref, pb_ref, xsa_ref, xsb_ref, dinv_ref, b_ref, g_ref, bt_ref):
    dinv = dinv_ref[...]
    psum = jnp.concatenate(
        [pa_ref[0:N, :] + pa_ref[R:R + N, :] + xsa_ref[...],
         pb_ref[0:N, :] + pb_ref[R:R + N, :] + xsb_ref[...]], axis=1)
    agg = dinv * psum + b_ref[...][None, :]
    mean = jnp.mean(agg, axis=0, keepdims=True)
    var = jnp.mean((agg - mean) ** 2, axis=0, keepdims=True)
    y = g_ref[...][None, :] * (agg - mean) * lax.rsqrt(var + EPS)
    return y + bt_ref[...][None, :], dinv


def _k_layer(pa, pb, xsa, xsb, dinv, b, g, bt, Wn, relu):
    def body(pa_ref, pb_ref, xsa_ref, xsb_ref, dinv_ref, b_ref, g_ref,
             bt_ref, w_ref, xsa_o, xsb_o):
        y, dinv = _agg_bn(pa_ref, pb_ref, xsa_ref, xsb_ref, dinv_ref,
                          b_ref, g_ref, bt_ref)
        if relu:
            y = jnp.maximum(y, 0.0)
        xw = jnp.dot(y, w_ref[...], preferred_element_type=jnp.float32)
        xs = dinv * xw
        xsa_o[...] = xs[:, 0:DW]
        xsb_o[...] = xs[:, DW:D]

    return pl.pallas_call(
        body,
        out_shape=(jax.ShapeDtypeStruct((N, DW), jnp.float32),
                   jax.ShapeDtypeStruct((N, DW), jnp.float32)),
    )(pa, pb, xsa, xsb, dinv, b, g, bt, Wn)


def _k_last(pa, pb, xsa, xsb, dinv, b, g, bt):
    def body(pa_ref, pb_ref, xsa_ref, xsb_ref, dinv_ref, b_ref, g_ref,
             bt_ref, out_ref):
        y, _ = _agg_bn(pa_ref, pb_ref, xsa_ref, xsb_ref, dinv_ref,
                       b_ref, g_ref, bt_ref)
        out_ref[...] = y

    return pl.pallas_call(
        body,
        out_shape=jax.ShapeDtypeStruct((N, D), jnp.float32),
    )(pa, pb, xsa, xsb, dinv, b, g, bt)


def _pad_cols(a, width):
    return jnp.concatenate(
        [a, jnp.zeros(a.shape[:-1] + (width - a.shape[-1],), a.dtype)], axis=-1)


def kernel(x, edge_index,
           We1, be1, g1, bt1, We2, be2, g2, bt2, We3, be3, g3, bt3,
           Wd1, bd1, gd1, btd1, Wd2, bd2, gd2, btd2, Wd3, bd3, gd3, btd3):
    row = edge_index[0].astype(jnp.int32)
    col = edge_index[1].astype(jnp.int32)
    pad = E_PAD - E
    r_idx = jnp.concatenate([row, jnp.zeros((pad,), jnp.int32)])
    c_idx = jnp.concatenate([col, jnp.full((pad,), JUNK, jnp.int32)])
    r_idx = r_idx.reshape(NC * NS * CH, CHUNK)
    c_idx = c_idx.reshape(NC * NS * CH, CHUNK)

    We3p = _pad_cols(We3, D)
    be3p = _pad_cols(be3, D)
    g3p = jnp.concatenate([g3, jnp.ones((D - g3.shape[0],), g3.dtype)])
    bt3p = _pad_cols(bt3, D)
    Wd1p = jnp.concatenate(
        [Wd1, jnp.zeros((D - Wd1.shape[0], Wd1.shape[1]), Wd1.dtype)], axis=0)

    hist = _sc_degree(c_idx)
    histT = jnp.transpose(hist)

    dinv, xsa, xsb = _k0(histT, x, We1)

    layers = [
        (be1, g1, bt1, We2, True),
        (be2, g2, bt2, We3p, True),
        (be3p, g3p, bt3p, Wd1p, False),
        (bd1, gd1, btd1, Wd2, True),
        (bd2, gd2, btd2, Wd3, True),
        (bd3, gd3, btd3, None, False),
    ]
    for b, g, bt, Wn, relu in layers:
        pa, pb = _sc_scatter(xsa, xsb, r_idx, c_idx)
        if Wn is None:
            return _k_last(pa, pb, xsa, xsb, dinv, b, g, bt)
        xsa, xsb = _k_layer(pa, pb, xsa, xsb, dinv, b, g, bt, Wn, relu)


# speedup vs baseline: 1.1335x; 1.0004x over previous
import functools

import jax
import jax.numpy as jnp
from jax import lax
from jax.experimental import pallas as pl
from jax.experimental.pallas import tpu as pltpu
from jax.experimental.pallas import tpu_sc as plsc

N = 10000
E = 320000
D = 128
DW = 64
NC = 2
NS = 16
CHUNK = 128
CH = 80
EPT = CH * CHUNK
E_PAD = NC * NS * EPT
RPT = 640
R = NS * RPT
JUNK = N
NBUF = 4
HALF = CH // 2
EPS = 1e-5

_MESH = plsc.VectorSubcoreMesh(core_axis_name="c", subcore_axis_name="s")


@functools.partial(
    pl.kernel,
    out_type=(jax.ShapeDtypeStruct((NC * R, DW), jnp.float32),
              jax.ShapeDtypeStruct((NC * R, DW), jnp.float32)),
    mesh=_MESH,
    scratch_types=[
        pltpu.VMEM((HALF, CHUNK), jnp.int32),
        pltpu.VMEM((HALF, CHUNK), jnp.int32),
        *[pltpu.VMEM((CHUNK, DW), jnp.float32) for _ in range(NBUF)],
        pltpu.VMEM_SHARED((N, DW), jnp.float32),
        pltpu.VMEM_SHARED((R, DW), jnp.float32),
        *[pltpu.SemaphoreType.DMA for _ in range(2 * NBUF)],
    ],
    compiler_params=pltpu.CompilerParams(use_tc_tiling_on_sc=False),
)
def _sc_scatter(srca_hbm, srcb_hbm, r_hbm, c_hbm, pa_hbm, pb_hbm,
                r_v, c_v, *rest):
    bufs = rest[:NBUF]
    xs_sp = rest[NBUF]
    acc = rest[NBUF + 1]
    gsem = rest[NBUF + 2:2 * NBUF + 2]
    ssem = rest[2 * NBUF + 2:]
    cid = lax.axis_index("c")
    sid = lax.axis_index("s")
    wid = cid * NS + sid

    for src_hbm, out_hbm in ((srca_hbm, pa_hbm), (srcb_hbm, pb_hbm)):
        @pl.when(sid == 0)
        def _stage():
            pltpu.sync_copy(src_hbm, xs_sp)

        z = bufs[0]

        def zrow(i, carry):
            for jj in range(DW // 16):
                z[i, pl.ds(jj * 16, 16)] = jnp.zeros((16,), jnp.float32)
            return carry

        lax.fori_loop(0, CHUNK, zrow, 0)
        for kk in range(RPT // CHUNK):
            pltpu.sync_copy(z, acc.at[pl.ds(sid * RPT + kk * CHUNK, CHUNK)])
        plsc.subcore_barrier()

        def g_start(j, b):
            pltpu.async_copy(xs_sp.at[r_v.at[j]], bufs[b], gsem[b])

        def g_wait(j, b):
            pltpu.make_async_copy(xs_sp.at[r_v.at[j]], bufs[b], gsem[b]).wait()

        def s_start(j, b):
            pltpu.async_copy(bufs[b], acc.at[c_v.at[j]], ssem[b], add=True)

        def s_wait(j, b):
            pltpu.make_async_copy(bufs[b], acc.at[c_v.at[j]], ssem[b]).wait()

        for h in range(2):
            base = wid * CH + h * HALF
            pltpu.sync_copy(r_hbm.at[pl.ds(base, HALF)], r_v)
            pltpu.sync_copy(c_hbm.at[pl.ds(base, HALF)], c_v)

            for m in range(NBUF):
                g_start(m, m)
            for j in range(2):
                g_wait(j, j)
                s_start(j, j)

            def step(t, carry):
                j0 = t * NBUF + 2
                for bp in range(NBUF):
                    j = j0 + bp
                    s_wait(j - 2, bp)
                    g_start(j + 2, bp)
                    b = (bp + 2) % NBUF
                    g_wait(j, b)
                    s_start(j, b)
                return carry

            lax.fori_loop(0, (HALF - 4) // NBUF, step, 0)
            for j in (HALF - 2, HALF - 1):
                s_wait(j - 2, (j - 2) % NBUF)
                g_wait(j, j % NBUF)
                s_start(j, j % NBUF)
            for j in (HALF - 2, HALF - 1):
                s_wait(j, j % NBUF)

        plsc.subcore_barrier()
        pltpu.sync_copy(acc.at[pl.ds(sid * RPT, RPT)],
                        out_hbm.at[pl.ds(cid * R + sid * RPT, RPT)])
        plsc.subcore_barrier()


@functools.partial(
    pl.kernel,
    out_type=jax.ShapeDtypeStruct((NC * NS, R), jnp.float32),
    mesh=_MESH,
    scratch_types=[
        pltpu.VMEM((CH, CHUNK), jnp.int32),
        pltpu.VMEM((R,), jnp.float32),
    ],
    compiler_params=pltpu.CompilerParams(needs_layout_passes=False),
)
def _sc_degree(c_hbm, out_hbm, c_v, hist):
    cid = lax.axis_index("c")
    sid = lax.axis_index("s")
    wid = cid * NS + sid
    pltpu.sync_copy(c_hbm.at[pl.ds(wid * CH, CH)], c_v)

    def zero(i, carry):
        hist[pl.ds(i * 16, 16)] = jnp.zeros((16,), jnp.float32)
        return carry

    lax.fori_loop(0, R // 16, zero, 0)
    ones = jnp.ones((16,), jnp.float32)

    def step(j, carry):
        for k in range(CHUNK // 16):
            idx = c_v[j, pl.ds(k * 16, 16)]
            plsc.addupdate_scatter(hist, [idx], ones)
        return carry

    lax.fori_loop(0, CH, step, 0)
    pltpu.sync_copy(hist, out_hbm.at[wid])


def _k_proj(x, W1):
    def body(x_ref, w_ref, xw_ref):
        xw_ref[...] = jnp.dot(x_ref[...], w_ref[...],
                              preferred_element_type=jnp.float32)

    return pl.pallas_call(
        body, out_shape=jax.ShapeDtypeStruct((N, D), jnp.float32))(x, W1)


def _k_dinv(histT, xw):
    def body(h_ref, xw_ref, dinv_ref, xsa_ref, xsb_ref):
        deg = jnp.sum(h_ref[...], axis=1, keepdims=True)
        dinv = lax.rsqrt(deg[0:N] + 1.0)
        dinv_ref[...] = dinv
        xs = dinv * xw_ref[...]
        xsa_ref[...] = xs[:, 0:DW]
        xsb_ref[...] = xs[:, DW:D]

    return pl.pallas_call(
        body,
        out_shape=(jax.ShapeDtypeStruct((N, 1), jnp.float32),
                   jax.ShapeDtypeStruct((N, DW), jnp.float32),
                   jax.ShapeDtypeStruct((N, DW), jnp.float32)),
    )(histT, xw)


def _agg_bn(pa_ref, pb_ref, xsa_ref, xsb_ref, dinv_ref, b_ref, g_ref, bt_ref):
    dinv = dinv_ref[...]
    psum = jnp.concatenate(
        [pa_ref[0:N, :] + pa_ref[R:R + N, :] + xsa_ref[...],
         pb_ref[0:N, :] + pb_ref[R:R + N, :] + xsb_ref[...]], axis=1)
    agg = dinv * psum + b_ref[...][None, :]
    mean = jnp.mean(agg, axis=0, keepdims=True)
    var = jnp.mean((agg - mean) ** 2, axis=0, keepdims=True)
    y = g_ref[...][None, :] * (agg - mean) * lax.rsqrt(var + EPS)
    return y + bt_ref[...][None, :], dinv


def _k_layer(pa, pb, xsa, xsb, dinv, b, g, bt, Wn, relu):
    def body(pa_ref, pb_ref, xsa_ref, xsb_ref, dinv_ref, b_ref, g_ref,
             bt_ref, w_ref, xsa_o, xsb_o):
        y, dinv = _agg_bn(pa_ref, pb_ref, xsa_ref, xsb_ref, dinv_ref,
                          b_ref, g_ref, bt_ref)
        if relu:
            y = jnp.maximum(y, 0.0)
        xw = jnp.dot(y, w_ref[...], preferred_element_type=jnp.float32)
        xs = dinv * xw
        xsa_o[...] = xs[:, 0:DW]
        xsb_o[...] = xs[:, DW:D]

    return pl.pallas_call(
        body,
        out_shape=(jax.ShapeDtypeStruct((N, DW), jnp.float32),
                   jax.ShapeDtypeStruct((N, DW), jnp.float32)),
    )(pa, pb, xsa, xsb, dinv, b, g, bt, Wn)


def _k_last(pa, pb, xsa, xsb, dinv, b, g, bt):
    def body(pa_ref, pb_ref, xsa_ref, xsb_ref, dinv_ref, b_ref, g_ref,
             bt_ref, out_ref):
        y, _ = _agg_bn(pa_ref, pb_ref, xsa_ref, xsb_ref, dinv_ref,
                       b_ref, g_ref, bt_ref)
        out_ref[...] = y

    return pl.pallas_call(
        body,
        out_shape=jax.ShapeDtypeStruct((N, D), jnp.float32),
    )(pa, pb, xsa, xsb, dinv, b, g, bt)


def _pad_cols(a, width):
    return jnp.concatenate(
        [a, jnp.zeros(a.shape[:-1] + (width - a.shape[-1],), a.dtype)], axis=-1)


def kernel(x, edge_index,
           We1, be1, g1, bt1, We2, be2, g2, bt2, We3, be3, g3, bt3,
           Wd1, bd1, gd1, btd1, Wd2, bd2, gd2, btd2, Wd3, bd3, gd3, btd3):
    row = edge_index[0].astype(jnp.int32)
    col = edge_index[1].astype(jnp.int32)
    pad = E_PAD - E
    r_idx = jnp.concatenate([row, jnp.zeros((pad,), jnp.int32)])
    c_idx = jnp.concatenate([col, jnp.full((pad,), JUNK, jnp.int32)])
    r_idx = r_idx.reshape(NC * NS * CH, CHUNK)
    c_idx = c_idx.reshape(NC * NS * CH, CHUNK)

    We3p = _pad_cols(We3, D)
    be3p = _pad_cols(be3, D)
    g3p = jnp.concatenate([g3, jnp.ones((D - g3.shape[0],), g3.dtype)])
    bt3p = _pad_cols(bt3, D)
    Wd1p = jnp.concatenate(
        [Wd1, jnp.zeros((D - Wd1.shape[0], Wd1.shape[1]), Wd1.dtype)], axis=0)

    hist = _sc_degree(c_idx)
    histT = jnp.transpose(hist)

    xw1 = _k_proj(x, We1)
    dinv, xsa, xsb = _k_dinv(histT, xw1)

    layers = [
        (be1, g1, bt1, We2, True),
        (be2, g2, bt2, We3p, True),
        (be3p, g3p, bt3p, Wd1p, False),
        (bd1, gd1, btd1, Wd2, True),
        (bd2, gd2, btd2, Wd3, True),
        (bd3, gd3, btd3, None, False),
    ]
    for b, g, bt, Wn, relu in layers:
        pa, pb = _sc_scatter(xsa, xsb, r_idx, c_idx)
        if Wn is None:
            return _k_last(pa, pb, xsa, xsb, dinv, b, g, bt)
        xsa, xsb = _k_layer(pa, pb, xsa, xsb, dinv, b, g, bt, Wn, relu)


# speedup vs baseline: 1.1530x; 1.0172x over previous
import functools

import jax
import jax.numpy as jnp
from jax import lax
from jax.experimental import pallas as pl
from jax.experimental.pallas import tpu as pltpu
from jax.experimental.pallas import tpu_sc as plsc

N = 10000
E = 320000
D = 128
DW = 64
NC = 2
NS = 16
CHUNK = 128
CH = 80
EPT = CH * CHUNK
E_PAD = NC * NS * EPT
RPT = 640
R = NS * RPT
JUNK = N
NBUF = 4
HALF = CH // 2
EPS = 1e-5

_MESH = plsc.VectorSubcoreMesh(core_axis_name="c", subcore_axis_name="s")


@functools.partial(
    pl.kernel,
    out_type=(jax.ShapeDtypeStruct((NC * R, DW), jnp.float32),
              jax.ShapeDtypeStruct((NC * R, DW), jnp.float32)),
    mesh=_MESH,
    scratch_types=[
        pltpu.VMEM((HALF, CHUNK), jnp.int32),
        pltpu.VMEM((HALF, CHUNK), jnp.int32),
        *[pltpu.VMEM((CHUNK, DW), jnp.float32) for _ in range(NBUF)],
        pltpu.VMEM_SHARED((N, DW), jnp.float32),
        pltpu.VMEM_SHARED((R, DW), jnp.float32),
        *[pltpu.SemaphoreType.DMA for _ in range(2 * NBUF)],
    ],
    compiler_params=pltpu.CompilerParams(use_tc_tiling_on_sc=False),
)
def _sc_scatter(srca_hbm, srcb_hbm, r_hbm, c_hbm, pa_hbm, pb_hbm,
                r_v, c_v, *rest):
    bufs = rest[:NBUF]
    xs_sp = rest[NBUF]
    acc = rest[NBUF + 1]
    gsem = rest[NBUF + 2:2 * NBUF + 2]
    ssem = rest[2 * NBUF + 2:]
    cid = lax.axis_index("c")
    sid = lax.axis_index("s")
    wid = cid * NS + sid

    NPT = N // NS

    for src_hbm, out_hbm in ((srca_hbm, pa_hbm), (srcb_hbm, pb_hbm)):
        base0 = wid * CH
        pltpu.async_copy(r_hbm.at[pl.ds(base0, HALF)], r_v, gsem[0])
        pltpu.async_copy(c_hbm.at[pl.ds(base0, HALF)], c_v, gsem[1])

        pltpu.sync_copy(src_hbm.at[pl.ds(sid * NPT, NPT)],
                        xs_sp.at[pl.ds(sid * NPT, NPT)])

        z = bufs[0]

        def zrow(i, carry):
            for jj in range(DW // 16):
                z[i, pl.ds(jj * 16, 16)] = jnp.zeros((16,), jnp.float32)
            return carry

        lax.fori_loop(0, CHUNK, zrow, 0)
        for kk in range(RPT // CHUNK):
            pltpu.sync_copy(z, acc.at[pl.ds(sid * RPT + kk * CHUNK, CHUNK)])
        plsc.subcore_barrier()

        def g_start(j, b):
            pltpu.async_copy(xs_sp.at[r_v.at[j]], bufs[b], gsem[b])

        def g_wait(j, b):
            pltpu.make_async_copy(xs_sp.at[r_v.at[j]], bufs[b], gsem[b]).wait()

        def s_start(j, b):
            pltpu.async_copy(bufs[b], acc.at[c_v.at[j]], ssem[b], add=True)

        def s_wait(j, b):
            pltpu.make_async_copy(bufs[b], acc.at[c_v.at[j]], ssem[b]).wait()

        for h in range(2):
            base = wid * CH + h * HALF
            if h == 0:
                pltpu.make_async_copy(
                    r_hbm.at[pl.ds(base, HALF)], r_v, gsem[0]).wait()
                pltpu.make_async_copy(
                    c_hbm.at[pl.ds(base, HALF)], c_v, gsem[1]).wait()
            else:
                pltpu.sync_copy(r_hbm.at[pl.ds(base, HALF)], r_v)
                pltpu.sync_copy(c_hbm.at[pl.ds(base, HALF)], c_v)

            for m in range(NBUF):
                g_start(m, m)
            for j in range(2):
                g_wait(j, j)
                s_start(j, j)

            def step(t, carry):
                j0 = t * NBUF + 2
                for bp in range(NBUF):
                    j = j0 + bp
                    s_wait(j - 2, bp)
                    g_start(j + 2, bp)
                    b = (bp + 2) % NBUF
                    g_wait(j, b)
                    s_start(j, b)
                return carry

            lax.fori_loop(0, (HALF - 4) // NBUF, step, 0)
            for j in (HALF - 2, HALF - 1):
                s_wait(j - 2, (j - 2) % NBUF)
                g_wait(j, j % NBUF)
                s_start(j, j % NBUF)
            for j in (HALF - 2, HALF - 1):
                s_wait(j, j % NBUF)

        plsc.subcore_barrier()
        pltpu.sync_copy(acc.at[pl.ds(sid * RPT, RPT)],
                        out_hbm.at[pl.ds(cid * R + sid * RPT, RPT)])
        plsc.subcore_barrier()


@functools.partial(
    pl.kernel,
    out_type=jax.ShapeDtypeStruct((NC * NS, R), jnp.float32),
    mesh=_MESH,
    scratch_types=[
        pltpu.VMEM((CH, CHUNK), jnp.int32),
        pltpu.VMEM((R,), jnp.float32),
    ],
    compiler_params=pltpu.CompilerParams(needs_layout_passes=False),
)
def _sc_degree(c_hbm, out_hbm, c_v, hist):
    cid = lax.axis_index("c")
    sid = lax.axis_index("s")
    wid = cid * NS + sid
    pltpu.sync_copy(c_hbm.at[pl.ds(wid * CH, CH)], c_v)

    def zero(i, carry):
        hist[pl.ds(i * 16, 16)] = jnp.zeros((16,), jnp.float32)
        return carry

    lax.fori_loop(0, R // 16, zero, 0)
    ones = jnp.ones((16,), jnp.float32)

    def step(j, carry):
        for k in range(CHUNK // 16):
            idx = c_v[j, pl.ds(k * 16, 16)]
            plsc.addupdate_scatter(hist, [idx], ones)
        return carry

    lax.fori_loop(0, CH, step, 0)
    pltpu.sync_copy(hist, out_hbm.at[wid])


def _k_proj(x, W1):
    def body(x_ref, w_ref, xw_ref):
        xw_ref[...] = jnp.dot(x_ref[...], w_ref[...],
                              preferred_element_type=jnp.float32)

    return pl.pallas_call(
        body, out_shape=jax.ShapeDtypeStruct((N, D), jnp.float32))(x, W1)


def _k_dinv(histT, xw):
    def body(h_ref, xw_ref, dinv_ref, xsa_ref, xsb_ref):
        deg = jnp.sum(h_ref[...], axis=1, keepdims=True)
        dinv = lax.rsqrt(deg[0:N] + 1.0)
        dinv_ref[...] = dinv
        xs = dinv * xw_ref[...]
        xsa_ref[...] = xs[:, 0:DW]
        xsb_ref[...] = xs[:, DW:D]

    return pl.pallas_call(
        body,
        out_shape=(jax.ShapeDtypeStruct((N, 1), jnp.float32),
                   jax.ShapeDtypeStruct((N, DW), jnp.float32),
                   jax.ShapeDtypeStruct((N, DW), jnp.float32)),
    )(histT, xw)


def _agg_bn(pa_ref, pb_ref, xsa_ref, xsb_ref, dinv_ref, b_ref, g_ref, bt_ref):
    dinv = dinv_ref[...]
    psum = jnp.concatenate(
        [pa_ref[0:N, :] + pa_ref[R:R + N, :] + xsa_ref[...],
         pb_ref[0:N, :] + pb_ref[R:R + N, :] + xsb_ref[...]], axis=1)
    agg = dinv * psum + b_ref[...][None, :]
    mean = jnp.mean(agg, axis=0, keepdims=True)
    var = jnp.mean((agg - mean) ** 2, axis=0, keepdims=True)
    y = g_ref[...][None, :] * (agg - mean) * lax.rsqrt(var + EPS)
    return y + bt_ref[...][None, :], dinv


def _k_layer(pa, pb, xsa, xsb, dinv, b, g, bt, Wn, relu):
    def body(pa_ref, pb_ref, xsa_ref, xsb_ref, dinv_ref, b_ref, g_ref,
             bt_ref, w_ref, xsa_o, xsb_o):
        y, dinv = _agg_bn(pa_ref, pb_ref, xsa_ref, xsb_ref, dinv_ref,
                          b_ref, g_ref, bt_ref)
        if relu:
            y = jnp.maximum(y, 0.0)
        xw = jnp.dot(y, w_ref[...], preferred_element_type=jnp.float32)
        xs = dinv * xw
        xsa_o[...] = xs[:, 0:DW]
        xsb_o[...] = xs[:, DW:D]

    return pl.pallas_call(
        body,
        out_shape=(jax.ShapeDtypeStruct((N, DW), jnp.float32),
                   jax.ShapeDtypeStruct((N, DW), jnp.float32)),
    )(pa, pb, xsa, xsb, dinv, b, g, bt, Wn)


def _k_last(pa, pb, xsa, xsb, dinv, b, g, bt):
    def body(pa_ref, pb_ref, xsa_ref, xsb_ref, dinv_ref, b_ref, g_ref,
             bt_ref, out_ref):
        y, _ = _agg_bn(pa_ref, pb_ref, xsa_ref, xsb_ref, dinv_ref,
                       b_ref, g_ref, bt_ref)
        out_ref[...] = y

    return pl.pallas_call(
        body,
        out_shape=jax.ShapeDtypeStruct((N, D), jnp.float32),
    )(pa, pb, xsa, xsb, dinv, b, g, bt)


def _pad_cols(a, width):
    return jnp.concatenate(
        [a, jnp.zeros(a.shape[:-1] + (width - a.shape[-1],), a.dtype)], axis=-1)


def kernel(x, edge_index,
           We1, be1, g1, bt1, We2, be2, g2, bt2, We3, be3, g3, bt3,
           Wd1, bd1, gd1, btd1, Wd2, bd2, gd2, btd2, Wd3, bd3, gd3, btd3):
    row = edge_index[0].astype(jnp.int32)
    col = edge_index[1].astype(jnp.int32)
    pad = E_PAD - E
    r_idx = jnp.concatenate([row, jnp.zeros((pad,), jnp.int32)])
    c_idx = jnp.concatenate([col, jnp.full((pad,), JUNK, jnp.int32)])
    r_idx = r_idx.reshape(NC * NS * CH, CHUNK)
    c_idx = c_idx.reshape(NC * NS * CH, CHUNK)

    We3p = _pad_cols(We3, D)
    be3p = _pad_cols(be3, D)
    g3p = jnp.concatenate([g3, jnp.ones((D - g3.shape[0],), g3.dtype)])
    bt3p = _pad_cols(bt3, D)
    Wd1p = jnp.concatenate(
        [Wd1, jnp.zeros((D - Wd1.shape[0], Wd1.shape[1]), Wd1.dtype)], axis=0)

    hist = _sc_degree(c_idx)
    histT = jnp.transpose(hist)

    xw1 = _k_proj(x, We1)
    dinv, xsa, xsb = _k_dinv(histT, xw1)

    layers = [
        (be1, g1, bt1, We2, True),
        (be2, g2, bt2, We3p, True),
        (be3p, g3p, bt3p, Wd1p, False),
        (bd1, gd1, btd1, Wd2, True),
        (bd2, gd2, btd2, Wd3, True),
        (bd3, gd3, btd3, None, False),
    ]
    for b, g, bt, Wn, relu in layers:
        pa, pb = _sc_scatter(xsa, xsb, r_idx, c_idx)
        if Wn is None:
            return _k_last(pa, pb, xsa, xsb, dinv, b, g, bt)
        xsa, xsb = _k_layer(pa, pb, xsa, xsb, dinv, b, g, bt, Wn, relu)


# speedup vs baseline: 1.2630x; 1.0955x over previous
import functools

import jax
import jax.numpy as jnp
from jax import lax
from jax.experimental import pallas as pl
from jax.experimental.pallas import tpu as pltpu
from jax.experimental.pallas import tpu_sc as plsc

N = 10000
E = 320000
D = 128
DW = 64
NC = 2
NS = 16
CHUNK = 128
CH = 80
EPT = CH * CHUNK
E_PAD = NC * NS * EPT
RPT = 640
R = NS * RPT
JUNK = N
NBUF = 4
HALF = CH // 2
EPS = 1e-5

_MESH = plsc.VectorSubcoreMesh(core_axis_name="c", subcore_axis_name="s")


@functools.partial(
    pl.kernel,
    out_type=(jax.ShapeDtypeStruct((NC * R, DW), jnp.float32),
              jax.ShapeDtypeStruct((NC * R, DW), jnp.float32)),
    mesh=_MESH,
    scratch_types=[
        pltpu.VMEM((HALF, CHUNK), jnp.int32),
        pltpu.VMEM((HALF, CHUNK), jnp.int32),
        *[pltpu.VMEM((CHUNK, DW), jnp.float32) for _ in range(NBUF)],
        pltpu.VMEM_SHARED((N, DW), jnp.float32),
        pltpu.VMEM_SHARED((R, DW), jnp.float32),
        *[pltpu.SemaphoreType.DMA for _ in range(2 * NBUF)],
    ],
    compiler_params=pltpu.CompilerParams(use_tc_tiling_on_sc=False),
)
def _sc_scatter(srca_hbm, srcb_hbm, r_hbm, c_hbm, pa_hbm, pb_hbm,
                r_v, c_v, *rest):
    bufs = rest[:NBUF]
    xs_sp = rest[NBUF]
    acc = rest[NBUF + 1]
    gsem = rest[NBUF + 2:2 * NBUF + 2]
    ssem = rest[2 * NBUF + 2:]
    cid = lax.axis_index("c")
    sid = lax.axis_index("s")
    wid = cid * NS + sid

    NPT = N // NS

    for src_hbm, out_hbm in ((srca_hbm, pa_hbm), (srcb_hbm, pb_hbm)):
        base0 = wid * CH
        pltpu.async_copy(r_hbm.at[pl.ds(base0, HALF)], r_v, gsem[0])
        pltpu.async_copy(c_hbm.at[pl.ds(base0, HALF)], c_v, gsem[1])

        pltpu.sync_copy(src_hbm.at[pl.ds(sid * NPT, NPT)],
                        xs_sp.at[pl.ds(sid * NPT, NPT)])

        z = bufs[0]

        def zrow(i, carry):
            for jj in range(DW // 16):
                z[i, pl.ds(jj * 16, 16)] = jnp.zeros((16,), jnp.float32)
            return carry

        lax.fori_loop(0, CHUNK, zrow, 0)
        for kk in range(RPT // CHUNK):
            pltpu.sync_copy(z, acc.at[pl.ds(sid * RPT + kk * CHUNK, CHUNK)])
        plsc.subcore_barrier()

        def g_start(j, b):
            pltpu.async_copy(xs_sp.at[r_v.at[j]], bufs[b], gsem[b])

        def g_wait(j, b):
            pltpu.make_async_copy(xs_sp.at[r_v.at[j]], bufs[b], gsem[b]).wait()

        def s_start(j, b):
            pltpu.async_copy(bufs[b], acc.at[c_v.at[j]], ssem[b], add=True)

        def s_wait(j, b):
            pltpu.make_async_copy(bufs[b], acc.at[c_v.at[j]], ssem[b]).wait()

        for h in range(2):
            base = wid * CH + h * HALF
            if h == 0:
                pltpu.make_async_copy(
                    r_hbm.at[pl.ds(base, HALF)], r_v, gsem[0]).wait()
                pltpu.make_async_copy(
                    c_hbm.at[pl.ds(base, HALF)], c_v, gsem[1]).wait()
            else:
                pltpu.sync_copy(r_hbm.at[pl.ds(base, HALF)], r_v)
                pltpu.sync_copy(c_hbm.at[pl.ds(base, HALF)], c_v)

            for m in range(NBUF):
                g_start(m, m)
            for j in range(2):
                g_wait(j, j)
                s_start(j, j)

            def step(t, carry):
                j0 = t * NBUF + 2
                for bp in range(NBUF):
                    j = j0 + bp
                    s_wait(j - 2, bp)
                    g_start(j + 2, bp)
                    b = (bp + 2) % NBUF
                    g_wait(j, b)
                    s_start(j, b)
                return carry

            lax.fori_loop(0, (HALF - 4) // NBUF, step, 0)
            for j in (HALF - 2, HALF - 1):
                s_wait(j - 2, (j - 2) % NBUF)
                g_wait(j, j % NBUF)
                s_start(j, j % NBUF)
            for j in (HALF - 2, HALF - 1):
                s_wait(j, j % NBUF)

        plsc.subcore_barrier()
        pltpu.sync_copy(acc.at[pl.ds(sid * RPT, RPT)],
                        out_hbm.at[pl.ds(cid * R + sid * RPT, RPT)])
        plsc.subcore_barrier()


@functools.partial(
    pl.kernel,
    out_type=jax.ShapeDtypeStruct((NC * NS, R), jnp.float32),
    mesh=_MESH,
    scratch_types=[
        pltpu.VMEM((CH, CHUNK), jnp.int32),
        pltpu.VMEM((R,), jnp.float32),
    ],
    compiler_params=pltpu.CompilerParams(needs_layout_passes=False),
)
def _sc_degree(c_hbm, out_hbm, c_v, hist):
    cid = lax.axis_index("c")
    sid = lax.axis_index("s")
    wid = cid * NS + sid
    pltpu.sync_copy(c_hbm.at[pl.ds(wid * CH, CH)], c_v)

    def zero(i, carry):
        hist[pl.ds(i * 16, 16)] = jnp.zeros((16,), jnp.float32)
        return carry

    lax.fori_loop(0, R // 16, zero, 0)
    ones = jnp.ones((16,), jnp.float32)

    def step(j, carry):
        for k in range(CHUNK // 16):
            idx = c_v[j, pl.ds(k * 16, 16)]
            plsc.addupdate_scatter(hist, [idx], ones)
        return carry

    lax.fori_loop(0, CH, step, 0)
    pltpu.sync_copy(hist, out_hbm.at[wid])


def _k_proj(x, W1):
    def body(x_ref, w_ref, xw_ref):
        xw_ref[...] = jnp.dot(x_ref[...], w_ref[...],
                              preferred_element_type=jnp.float32)

    return pl.pallas_call(
        body, out_shape=jax.ShapeDtypeStruct((N, D), jnp.float32))(x, W1)


def _k_dinv(histT, xw):
    def body(h_ref, xw_ref, dinv_ref, xsa_ref, xsb_ref):
        deg = jnp.sum(h_ref[...], axis=1, keepdims=True)
        dinv = lax.rsqrt(deg[0:N] + 1.0)
        dinv_ref[...] = dinv
        xs = dinv * xw_ref[...]
        xsa_ref[...] = xs[:, 0:DW]
        xsb_ref[...] = xs[:, DW:D]

    return pl.pallas_call(
        body,
        out_shape=(jax.ShapeDtypeStruct((N, 1), jnp.float32),
                   jax.ShapeDtypeStruct((N, DW), jnp.float32),
                   jax.ShapeDtypeStruct((N, DW), jnp.float32)),
    )(histT, xw)


def _unpack(p_ref):
    ps = p_ref[0:N // 2, :] + p_ref[R // 2:(R + N) // 2, :]
    return jnp.concatenate([ps[:, 0:DW], ps[:, DW:D]], axis=0)


def _agg_bn(pa_ref, pb_ref, xsa_ref, xsb_ref, dinv_ref, b_ref, g_ref, bt_ref):
    dinv = dinv_ref[...]
    psum = jnp.concatenate(
        [_unpack(pa_ref) + xsa_ref[...],
         _unpack(pb_ref) + xsb_ref[...]], axis=1)
    agg = dinv * psum + b_ref[...][None, :]
    mean = jnp.mean(agg, axis=0, keepdims=True)
    var = jnp.mean((agg - mean) ** 2, axis=0, keepdims=True)
    y = g_ref[...][None, :] * (agg - mean) * lax.rsqrt(var + EPS)
    return y + bt_ref[...][None, :], dinv


def _k_layer(pa, pb, xsa, xsb, dinv, b, g, bt, Wn, relu):
    def body(pa_ref, pb_ref, xsa_ref, xsb_ref, dinv_ref, b_ref, g_ref,
             bt_ref, w_ref, xsa_o, xsb_o):
        y, dinv = _agg_bn(pa_ref, pb_ref, xsa_ref, xsb_ref, dinv_ref,
                          b_ref, g_ref, bt_ref)
        if relu:
            y = jnp.maximum(y, 0.0)
        xw = jnp.dot(y, w_ref[...], preferred_element_type=jnp.float32)
        xs = dinv * xw
        xsa_o[...] = xs[:, 0:DW]
        xsb_o[...] = xs[:, DW:D]

    return pl.pallas_call(
        body,
        out_shape=(jax.ShapeDtypeStruct((N, DW), jnp.float32),
                   jax.ShapeDtypeStruct((N, DW), jnp.float32)),
    )(pa, pb, xsa, xsb, dinv, b, g, bt, Wn)


def _k_last(pa, pb, xsa, xsb, dinv, b, g, bt):
    def body(pa_ref, pb_ref, xsa_ref, xsb_ref, dinv_ref, b_ref, g_ref,
             bt_ref, out_ref):
        y, _ = _agg_bn(pa_ref, pb_ref, xsa_ref, xsb_ref, dinv_ref,
                       b_ref, g_ref, bt_ref)
        out_ref[...] = y

    return pl.pallas_call(
        body,
        out_shape=jax.ShapeDtypeStruct((N, D), jnp.float32),
    )(pa, pb, xsa, xsb, dinv, b, g, bt)


def _pad_cols(a, width):
    return jnp.concatenate(
        [a, jnp.zeros(a.shape[:-1] + (width - a.shape[-1],), a.dtype)], axis=-1)


def kernel(x, edge_index,
           We1, be1, g1, bt1, We2, be2, g2, bt2, We3, be3, g3, bt3,
           Wd1, bd1, gd1, btd1, Wd2, bd2, gd2, btd2, Wd3, bd3, gd3, btd3):
    row = edge_index[0].astype(jnp.int32)
    col = edge_index[1].astype(jnp.int32)
    pad = E_PAD - E
    r_idx = jnp.concatenate([row, jnp.zeros((pad,), jnp.int32)])
    c_idx = jnp.concatenate([col, jnp.full((pad,), JUNK, jnp.int32)])
    c_map = jnp.where(c_idx < N // 2, 2 * c_idx,
                      jnp.where(c_idx < N, 2 * c_idx - (N - 1), c_idx))
    r_idx = r_idx.reshape(NC * NS * CH, CHUNK)
    c_idx = c_idx.reshape(NC * NS * CH, CHUNK)
    c_map = c_map.reshape(NC * NS * CH, CHUNK)

    We3p = _pad_cols(We3, D)
    be3p = _pad_cols(be3, D)
    g3p = jnp.concatenate([g3, jnp.ones((D - g3.shape[0],), g3.dtype)])
    bt3p = _pad_cols(bt3, D)
    Wd1p = jnp.concatenate(
        [Wd1, jnp.zeros((D - Wd1.shape[0], Wd1.shape[1]), Wd1.dtype)], axis=0)

    hist = _sc_degree(c_idx)
    histT = jnp.transpose(hist)

    xw1 = _k_proj(x, We1)
    dinv, xsa, xsb = _k_dinv(histT, xw1)

    layers = [
        (be1, g1, bt1, We2, True),
        (be2, g2, bt2, We3p, True),
        (be3p, g3p, bt3p, Wd1p, False),
        (bd1, gd1, btd1, Wd2, True),
        (bd2, gd2, btd2, Wd3, True),
        (bd3, gd3, btd3, None, False),
    ]
    for b, g, bt, Wn, relu in layers:
        pa, pb = _sc_scatter(xsa, xsb, r_idx, c_map)
        pa = jnp.reshape(pa, (NC * R // 2, D))
        pb = jnp.reshape(pb, (NC * R // 2, D))
        if Wn is None:
            return _k_last(pa, pb, xsa, xsb, dinv, b, g, bt)
        xsa, xsb = _k_layer(pa, pb, xsa, xsb, dinv, b, g, bt, Wn, relu)


# speedup vs baseline: 1.3426x; 1.0630x over previous
import functools

import jax
import jax.numpy as jnp
from jax import lax
from jax.experimental import pallas as pl
from jax.experimental.pallas import tpu as pltpu
from jax.experimental.pallas import tpu_sc as plsc

N = 10000
E = 320000
D = 128
DW = 64
NC = 2
NS = 16
CHUNK = 128
CH = 80
EPT = CH * CHUNK
E_PAD = NC * NS * EPT
RPT = 640
R = NS * RPT
JUNK = N
NBUF = 4
HALF = CH // 2
EPS = 1e-5

_MESH = plsc.VectorSubcoreMesh(core_axis_name="c", subcore_axis_name="s")


@functools.partial(
    pl.kernel,
    out_type=(jax.ShapeDtypeStruct((NC * R, DW), jnp.float32),
              jax.ShapeDtypeStruct((NC * R, DW), jnp.float32)),
    mesh=_MESH,
    scratch_types=[
        pltpu.VMEM((HALF, CHUNK), jnp.int32),
        pltpu.VMEM((HALF, CHUNK), jnp.int32),
        *[pltpu.VMEM((CHUNK, DW), jnp.float32) for _ in range(NBUF)],
        pltpu.VMEM_SHARED((N, DW), jnp.float32),
        pltpu.VMEM_SHARED((R, DW), jnp.float32),
        *[pltpu.SemaphoreType.DMA for _ in range(2 * NBUF)],
    ],
    compiler_params=pltpu.CompilerParams(use_tc_tiling_on_sc=False),
)
def _sc_scatter(srca_hbm, srcb_hbm, r_hbm, c_hbm, pa_hbm, pb_hbm,
                r_v, c_v, *rest):
    bufs = rest[:NBUF]
    xs_sp = rest[NBUF]
    acc = rest[NBUF + 1]
    gsem = rest[NBUF + 2:2 * NBUF + 2]
    ssem = rest[2 * NBUF + 2:]
    cid = lax.axis_index("c")
    sid = lax.axis_index("s")
    wid = cid * NS + sid

    NPT = N // NS

    for src_hbm, out_hbm in ((srca_hbm, pa_hbm), (srcb_hbm, pb_hbm)):
        base0 = wid * CH
        pltpu.async_copy(r_hbm.at[pl.ds(base0, HALF)], r_v, gsem[0])
        pltpu.async_copy(c_hbm.at[pl.ds(base0, HALF)], c_v, gsem[1])

        pltpu.sync_copy(src_hbm.at[pl.ds(sid * NPT, NPT)],
                        xs_sp.at[pl.ds(sid * NPT, NPT)])

        z = bufs[0]

        def zrow(i, carry):
            for jj in range(DW // 16):
                z[i, pl.ds(jj * 16, 16)] = jnp.zeros((16,), jnp.float32)
            return carry

        lax.fori_loop(0, CHUNK, zrow, 0)
        for kk in range(RPT // CHUNK):
            pltpu.sync_copy(z, acc.at[pl.ds(sid * RPT + kk * CHUNK, CHUNK)])
        plsc.subcore_barrier()

        def g_start(j, b):
            pltpu.async_copy(xs_sp.at[r_v.at[j]], bufs[b], gsem[b])

        def g_wait(j, b):
            pltpu.make_async_copy(xs_sp.at[r_v.at[j]], bufs[b], gsem[b]).wait()

        def s_start(j, b):
            pltpu.async_copy(bufs[b], acc.at[c_v.at[j]], ssem[b], add=True)

        def s_wait(j, b):
            pltpu.make_async_copy(bufs[b], acc.at[c_v.at[j]], ssem[b]).wait()

        for h in range(2):
            base = wid * CH + h * HALF
            if h == 0:
                pltpu.make_async_copy(
                    r_hbm.at[pl.ds(base, HALF)], r_v, gsem[0]).wait()
                pltpu.make_async_copy(
                    c_hbm.at[pl.ds(base, HALF)], c_v, gsem[1]).wait()
            else:
                pltpu.sync_copy(r_hbm.at[pl.ds(base, HALF)], r_v)
                pltpu.sync_copy(c_hbm.at[pl.ds(base, HALF)], c_v)

            for m in range(NBUF):
                g_start(m, m)
            for j in range(2):
                g_wait(j, j)
                s_start(j, j)

            def step(t, carry):
                j0 = t * NBUF + 2
                for bp in range(NBUF):
                    j = j0 + bp
                    s_wait(j - 2, bp)
                    g_start(j + 2, bp)
                    b = (bp + 2) % NBUF
                    g_wait(j, b)
                    s_start(j, b)
                return carry

            lax.fori_loop(0, (HALF - 4) // NBUF, step, 0)
            for j in (HALF - 2, HALF - 1):
                s_wait(j - 2, (j - 2) % NBUF)
                g_wait(j, j % NBUF)
                s_start(j, j % NBUF)
            for j in (HALF - 2, HALF - 1):
                s_wait(j, j % NBUF)

        plsc.subcore_barrier()
        pltpu.sync_copy(acc.at[pl.ds(sid * RPT, RPT)],
                        out_hbm.at[pl.ds(cid * R + sid * RPT, RPT)])
        plsc.subcore_barrier()


@functools.partial(
    pl.kernel,
    out_type=jax.ShapeDtypeStruct((NC * NS, R), jnp.float32),
    mesh=_MESH,
    scratch_types=[
        pltpu.VMEM((CH, CHUNK), jnp.int32),
        pltpu.VMEM((R,), jnp.float32),
    ],
    compiler_params=pltpu.CompilerParams(needs_layout_passes=False),
)
def _sc_degree(c_hbm, out_hbm, c_v, hist):
    cid = lax.axis_index("c")
    sid = lax.axis_index("s")
    wid = cid * NS + sid
    pltpu.sync_copy(c_hbm.at[pl.ds(wid * CH, CH)], c_v)

    def zero(i, carry):
        hist[pl.ds(i * 16, 16)] = jnp.zeros((16,), jnp.float32)
        return carry

    lax.fori_loop(0, R // 16, zero, 0)
    ones = jnp.ones((16,), jnp.float32)

    def step(j, carry):
        for k in range(CHUNK // 16):
            idx = c_v[j, pl.ds(k * 16, 16)]
            plsc.addupdate_scatter(hist, [idx], ones)
        return carry

    lax.fori_loop(0, CH, step, 0)
    pltpu.sync_copy(hist, out_hbm.at[wid])


def _k_proj(x, W1):
    def body(x_ref, w_ref, xw_ref):
        xw_ref[...] = jnp.dot(x_ref[...], w_ref[...],
                              preferred_element_type=jnp.float32)

    return pl.pallas_call(
        body, out_shape=jax.ShapeDtypeStruct((N, D), jnp.float32))(x, W1)


def _k_dinv(histT, xw):
    def body(h_ref, xw_ref, dinv_ref, xsa_ref, xsb_ref):
        deg = jnp.sum(h_ref[...], axis=1, keepdims=True)
        dinv = lax.rsqrt(deg[0:N] + 1.0)
        dinv_ref[...] = dinv
        xsa_ref[...], xsb_ref[...] = _pack(dinv * xw_ref[...])

    return pl.pallas_call(
        body,
        out_shape=(jax.ShapeDtypeStruct((N, 1), jnp.float32),
                   jax.ShapeDtypeStruct((N // 2, D), jnp.float32),
                   jax.ShapeDtypeStruct((N // 2, D), jnp.float32)),
    )(histT, xw)


def _unpack(x):
    return jnp.concatenate([x[:, 0:DW], x[:, DW:D]], axis=0)


def _pack(xs):
    def half(lo, hi):
        return jnp.concatenate([xs[0:N // 2, lo:hi], xs[N // 2:N, lo:hi]],
                               axis=1)
    return half(0, DW), half(DW, D)


def _agg_bn(pa_ref, pb_ref, xsa_ref, xsb_ref, dinv_ref, b_ref, g_ref, bt_ref):
    dinv = dinv_ref[...]
    pa = pa_ref[0:N // 2, :] + pa_ref[R // 2:(R + N) // 2, :]
    pb = pb_ref[0:N // 2, :] + pb_ref[R // 2:(R + N) // 2, :]
    psum = jnp.concatenate(
        [_unpack(pa) + _unpack(xsa_ref[...]),
         _unpack(pb) + _unpack(xsb_ref[...])], axis=1)
    agg = dinv * psum + b_ref[...][None, :]
    mean = jnp.mean(agg, axis=0, keepdims=True)
    var = jnp.mean((agg - mean) ** 2, axis=0, keepdims=True)
    y = g_ref[...][None, :] * (agg - mean) * lax.rsqrt(var + EPS)
    return y + bt_ref[...][None, :], dinv


def _k_layer(pa, pb, xsa, xsb, dinv, b, g, bt, Wn, relu):
    def body(pa_ref, pb_ref, xsa_ref, xsb_ref, dinv_ref, b_ref, g_ref,
             bt_ref, w_ref, xsa_o, xsb_o):
        y, dinv = _agg_bn(pa_ref, pb_ref, xsa_ref, xsb_ref, dinv_ref,
                          b_ref, g_ref, bt_ref)
        if relu:
            y = jnp.maximum(y, 0.0)
        xw = jnp.dot(y, w_ref[...], preferred_element_type=jnp.float32)
        xsa_o[...], xsb_o[...] = _pack(dinv * xw)

    return pl.pallas_call(
        body,
        out_shape=(jax.ShapeDtypeStruct((N // 2, D), jnp.float32),
                   jax.ShapeDtypeStruct((N // 2, D), jnp.float32)),
    )(pa, pb, xsa, xsb, dinv, b, g, bt, Wn)


def _k_last(pa, pb, xsa, xsb, dinv, b, g, bt):
    def body(pa_ref, pb_ref, xsa_ref, xsb_ref, dinv_ref, b_ref, g_ref,
             bt_ref, out_ref):
        y, _ = _agg_bn(pa_ref, pb_ref, xsa_ref, xsb_ref, dinv_ref,
                       b_ref, g_ref, bt_ref)
        out_ref[...] = y

    return pl.pallas_call(
        body,
        out_shape=jax.ShapeDtypeStruct((N, D), jnp.float32),
    )(pa, pb, xsa, xsb, dinv, b, g, bt)


def _pad_cols(a, width):
    return jnp.concatenate(
        [a, jnp.zeros(a.shape[:-1] + (width - a.shape[-1],), a.dtype)], axis=-1)


def kernel(x, edge_index,
           We1, be1, g1, bt1, We2, be2, g2, bt2, We3, be3, g3, bt3,
           Wd1, bd1, gd1, btd1, Wd2, bd2, gd2, btd2, Wd3, bd3, gd3, btd3):
    row = edge_index[0].astype(jnp.int32)
    col = edge_index[1].astype(jnp.int32)
    pad = E_PAD - E
    r_idx = jnp.concatenate([row, jnp.zeros((pad,), jnp.int32)])
    c_idx = jnp.concatenate([col, jnp.full((pad,), JUNK, jnp.int32)])
    c_map = jnp.where(c_idx < N // 2, 2 * c_idx,
                      jnp.where(c_idx < N, 2 * c_idx - (N - 1), c_idx))
    r_map = jnp.where(r_idx < N // 2, 2 * r_idx, 2 * r_idx - (N - 1))
    c_idx = c_idx.reshape(NC * NS * CH, CHUNK)
    c_map = c_map.reshape(NC * NS * CH, CHUNK)
    r_map = r_map.reshape(NC * NS * CH, CHUNK)

    We3p = _pad_cols(We3, D)
    be3p = _pad_cols(be3, D)
    g3p = jnp.concatenate([g3, jnp.ones((D - g3.shape[0],), g3.dtype)])
    bt3p = _pad_cols(bt3, D)
    Wd1p = jnp.concatenate(
        [Wd1, jnp.zeros((D - Wd1.shape[0], Wd1.shape[1]), Wd1.dtype)], axis=0)

    hist = _sc_degree(c_idx)
    histT = jnp.transpose(hist)

    xw1 = _k_proj(x, We1)
    dinv, xsa, xsb = _k_dinv(histT, xw1)

    layers = [
        (be1, g1, bt1, We2, True),
        (be2, g2, bt2, We3p, True),
        (be3p, g3p, bt3p, Wd1p, False),
        (bd1, gd1, btd1, Wd2, True),
        (bd2, gd2, btd2, Wd3, True),
        (bd3, gd3, btd3, None, False),
    ]
    for b, g, bt, Wn, relu in layers:
        pa, pb = _sc_scatter(jnp.reshape(xsa, (N, DW)),
                             jnp.reshape(xsb, (N, DW)), r_map, c_map)
        pa = jnp.reshape(pa, (NC * R // 2, D))
        pb = jnp.reshape(pb, (NC * R // 2, D))
        if Wn is None:
            return _k_last(pa, pb, xsa, xsb, dinv, b, g, bt)
        xsa, xsb = _k_layer(pa, pb, xsa, xsb, dinv, b, g, bt, Wn, relu)
